# baseline (speedup 1.0000x reference)
import functools
from math import log

import jax
import jax.numpy as jnp
from jax import lax
from jax.experimental import pallas as pl
from jax.experimental.pallas import tpu as pltpu
from jax.experimental.pallas import tpu_sc as plsc

N_NODES = 10000
N_EDGES = 320000
DIM_NODE = 128
DIM_HIDDEN = 256
HALF = DIM_HIDDEN // 2
NUM_CLASSES = 40
ALPHA = 0.1
THETA = 0.5
NUM_LAYERS = 8

NC = 2
NS = 16
NPAD = 10240
ROWS_PER_TILE = NPAD // NS
ZROWS = 128
DEGW = 16

BLK = 128
BATCH = 8
EPAD = 327680
NBLK_TILE = EPAD // (NS * BLK)
NBATCH = NBLK_TILE // BATCH
NBLK_DEG = EPAD // (NC * NS * BLK)
NBATCH_DEG = NBLK_DEG // BATCH
NTRASH = 240

_MESH = plsc.VectorSubcoreMesh(core_axis_name="c", subcore_axis_name="s")


def _zero_fill(ref, nrows, width):
    def row(i, _):
        for j in range(width // 16):
            ref[i, pl.ds(j * 16, 16)] = jnp.zeros((16,), jnp.float32)
        return 0
    lax.fori_loop(0, nrows, row, 0)


def _zero_acc(zbuf_v, acc_sh, s, width):
    _zero_fill(zbuf_v, ZROWS, width)
    for k in range(ROWS_PER_TILE // ZROWS):
        pltpu.sync_copy(zbuf_v, acc_sh.at[pl.ds(s * ROWS_PER_TILE + k * ZROWS, ZROWS)])


def _deg_body(dst2d_hbm, deg0_hbm, deg1_hbm, ones_v, zbuf_v, idx_v, acc_sh):
    c = lax.axis_index("c")
    s = lax.axis_index("s")
    wid = s * NC + c

    def fill_ones(i, _):
        ones_v[i, :] = jnp.ones((16,), jnp.float32)
        return 0
    lax.fori_loop(0, BLK, fill_ones, 0)

    _zero_acc(zbuf_v, acc_sh, s, DEGW)
    plsc.subcore_barrier()

    blk0 = wid * NBLK_DEG
    def batch(bi, _):
        pltpu.sync_copy(dst2d_hbm.at[pl.ds(blk0 + bi * BATCH, BATCH)], idx_v)
        for j in range(BATCH):
            pltpu.sync_copy(ones_v, acc_sh.at[idx_v.at[j]], add=True)
        return 0
    lax.fori_loop(0, NBATCH_DEG, batch, 0)
    plsc.subcore_barrier()

    rows = pl.ds(s * ROWS_PER_TILE, ROWS_PER_TILE)
    @pl.when(c == 0)
    def _():
        pltpu.sync_copy(acc_sh.at[rows], deg0_hbm.at[rows])
    @pl.when(c == 1)
    def _():
        pltpu.sync_copy(acc_sh.at[rows], deg1_hbm.at[rows])


_sc_deg = functools.partial(
    pl.kernel,
    out_type=(
        jax.ShapeDtypeStruct((NPAD, DEGW), jnp.float32),
        jax.ShapeDtypeStruct((NPAD, DEGW), jnp.float32),
    ),
    mesh=_MESH,
    scratch_types=[
        pltpu.VMEM((BLK, DEGW), jnp.float32),
        pltpu.VMEM((ZROWS, DEGW), jnp.float32),
        pltpu.VMEM((BATCH, BLK), jnp.int32),
        pltpu.VMEM_SHARED((NPAD, DEGW), jnp.float32),
    ],
)(_deg_body)


def _msg_body(g0_hbm, g1_hbm, src2d_hbm, dst2d_hbm, agg0_hbm, agg1_hbm,
              idxs_v, idxd_v, rows0_v, rows1_v, zbuf_v, acc_sh, semg0, semg1):
    c = lax.axis_index("c")
    s = lax.axis_index("s")

    _zero_fill(zbuf_v, 64, HALF)
    for k in range(ROWS_PER_TILE // 64):
        pltpu.sync_copy(zbuf_v, acc_sh.at[pl.ds(s * ROWS_PER_TILE + k * 64, 64)])
    plsc.subcore_barrier()

    blk0 = s * NBLK_TILE

    def run(g_hbm):
        def batch(bi, _):
            row0 = blk0 + bi * BATCH
            pltpu.sync_copy(src2d_hbm.at[pl.ds(row0, BATCH)], idxs_v)
            pltpu.sync_copy(dst2d_hbm.at[pl.ds(row0, BATCH)], idxd_v)
            bufs = [rows0_v, rows1_v]
            sems = [semg0, semg1]
            d = [None] * BATCH
            d[0] = pltpu.async_copy(g_hbm.at[idxs_v.at[0]], bufs[0], sems[0])
            for j in range(1, BATCH):
                d[j] = pltpu.async_copy(
                    g_hbm.at[idxs_v.at[j]], bufs[j % 2], sems[j % 2])
                d[j - 1].wait()
                pltpu.sync_copy(bufs[(j - 1) % 2],
                                acc_sh.at[idxd_v.at[j - 1]], add=True)
            d[BATCH - 1].wait()
            pltpu.sync_copy(bufs[(BATCH - 1) % 2],
                            acc_sh.at[idxd_v.at[BATCH - 1]], add=True)
            return 0
        lax.fori_loop(0, NBATCH, batch, 0)

    @pl.when(c == 0)
    def _():
        run(g0_hbm)
    @pl.when(c == 1)
    def _():
        run(g1_hbm)
    plsc.subcore_barrier()

    rows = pl.ds(s * ROWS_PER_TILE, ROWS_PER_TILE)
    @pl.when(c == 0)
    def _():
        pltpu.sync_copy(acc_sh.at[rows], agg0_hbm.at[rows])
    @pl.when(c == 1)
    def _():
        pltpu.sync_copy(acc_sh.at[rows], agg1_hbm.at[rows])


_sc_msg = functools.partial(
    pl.kernel,
    out_type=(
        jax.ShapeDtypeStruct((NPAD, HALF), jnp.float32),
        jax.ShapeDtypeStruct((NPAD, HALF), jnp.float32),
    ),
    mesh=_MESH,
    scratch_types=[
        pltpu.VMEM((BATCH, BLK), jnp.int32),
        pltpu.VMEM((BATCH, BLK), jnp.int32),
        pltpu.VMEM((BLK, HALF), jnp.float32),
        pltpu.VMEM((BLK, HALF), jnp.float32),
        pltpu.VMEM((64, HALF), jnp.float32),
        pltpu.VMEM_SHARED((NPAD, HALF), jnp.float32),
        pltpu.SemaphoreType.DMA,
        pltpu.SemaphoreType.DMA,
    ],
)(_msg_body)


_RB = 1000
_GRID = N_NODES // _RB


def _fc1_kernel(x_ref, w_ref, b_ref, d0_ref, d1_ref,
                h0_ref, g0_ref, g1_ref, dinv_ref):
    h = jnp.maximum(
        jnp.dot(x_ref[...], w_ref[...], preferred_element_type=jnp.float32)
        + b_ref[...], 0.0)
    deg = 1.0 + d0_ref[:, 0:1] + d1_ref[:, 0:1]
    dinv = lax.rsqrt(deg)
    h0_ref[...] = h
    g0_ref[...] = dinv * h[:, :HALF]
    g1_ref[...] = dinv * h[:, HALF:]
    dinv_ref[...] = jnp.broadcast_to(dinv, (_RB, HALF))


def _tc_fc1(x, fc1_w, fc1_b, deg0, deg1):
    return pl.pallas_call(
        _fc1_kernel,
        grid=(_GRID,),
        in_specs=[
            pl.BlockSpec((_RB, DIM_NODE), lambda b: (b, 0)),
            pl.BlockSpec((DIM_NODE, DIM_HIDDEN), lambda b: (0, 0)),
            pl.BlockSpec((1, DIM_HIDDEN), lambda b: (0, 0)),
            pl.BlockSpec((_RB, DEGW), lambda b: (b, 0)),
            pl.BlockSpec((_RB, DEGW), lambda b: (b, 0)),
        ],
        out_specs=[
            pl.BlockSpec((_RB, DIM_HIDDEN), lambda b: (b, 0)),
            pl.BlockSpec((_RB, HALF), lambda b: (b, 0)),
            pl.BlockSpec((_RB, HALF), lambda b: (b, 0)),
            pl.BlockSpec((_RB, HALF), lambda b: (b, 0)),
        ],
        out_shape=[
            jax.ShapeDtypeStruct((N_NODES, DIM_HIDDEN), jnp.float32),
            jax.ShapeDtypeStruct((N_NODES, HALF), jnp.float32),
            jax.ShapeDtypeStruct((N_NODES, HALF), jnp.float32),
            jax.ShapeDtypeStruct((N_NODES, HALF), jnp.float32),
        ],
    )(x, fc1_w, fc1_b, deg0, deg1)


def _layer_kernel(beta, ag0_ref, ag1_ref, g0_ref, g1_ref, h0_ref, dinv_ref,
                  w_ref, hn_ref, g0n_ref, g1n_ref):
    dinv = dinv_ref[...]
    a0 = dinv * (ag0_ref[...] + g0_ref[...])
    a1 = dinv * (ag1_ref[...] + g1_ref[...])
    z = (1.0 - ALPHA) * jnp.concatenate([a0, a1], axis=1) + ALPHA * h0_ref[...]
    out = (1.0 - beta) * z + beta * jnp.dot(
        z, w_ref[...], preferred_element_type=jnp.float32)
    h = jnp.maximum(out, 0.0)
    hn_ref[...] = h
    g0n_ref[...] = dinv * h[:, :HALF]
    g1n_ref[...] = dinv * h[:, HALF:]


def _tc_layer(beta, ag0, ag1, g0, g1, h0, dinv, w):
    return pl.pallas_call(
        functools.partial(_layer_kernel, beta),
        grid=(_GRID,),
        in_specs=[
            pl.BlockSpec((_RB, HALF), lambda b: (b, 0)),
            pl.BlockSpec((_RB, HALF), lambda b: (b, 0)),
            pl.BlockSpec((_RB, HALF), lambda b: (b, 0)),
            pl.BlockSpec((_RB, HALF), lambda b: (b, 0)),
            pl.BlockSpec((_RB, DIM_HIDDEN), lambda b: (b, 0)),
            pl.BlockSpec((_RB, HALF), lambda b: (b, 0)),
            pl.BlockSpec((DIM_HIDDEN, DIM_HIDDEN), lambda b: (0, 0)),
        ],
        out_specs=[
            pl.BlockSpec((_RB, DIM_HIDDEN), lambda b: (b, 0)),
            pl.BlockSpec((_RB, HALF), lambda b: (b, 0)),
            pl.BlockSpec((_RB, HALF), lambda b: (b, 0)),
        ],
        out_shape=[
            jax.ShapeDtypeStruct((N_NODES, DIM_HIDDEN), jnp.float32),
            jax.ShapeDtypeStruct((N_NODES, HALF), jnp.float32),
            jax.ShapeDtypeStruct((N_NODES, HALF), jnp.float32),
        ],
    )(ag0, ag1, g0, g1, h0, dinv, w)


def _fc2_kernel(h_ref, w_ref, b_ref, o_ref):
    o_ref[...] = jnp.dot(h_ref[...], w_ref[...],
                         preferred_element_type=jnp.float32) + b_ref[...]


def _tc_fc2(h, fc2_w, fc2_b):
    return pl.pallas_call(
        _fc2_kernel,
        grid=(_GRID,),
        in_specs=[
            pl.BlockSpec((_RB, DIM_HIDDEN), lambda b: (b, 0)),
            pl.BlockSpec((DIM_HIDDEN, NUM_CLASSES), lambda b: (0, 0)),
            pl.BlockSpec((1, NUM_CLASSES), lambda b: (0, 0)),
        ],
        out_specs=pl.BlockSpec((_RB, NUM_CLASSES), lambda b: (b, 0)),
        out_shape=jax.ShapeDtypeStruct((N_NODES, NUM_CLASSES), jnp.float32),
    )(h, fc2_w, fc2_b)


def kernel(x, edge_index, fc1_w, fc1_b, conv_ws, fc2_w, fc2_b):
    src = edge_index[0]
    dst = edge_index[1]
    npad_e = EPAD - N_EDGES
    pad_src = jnp.arange(npad_e, dtype=jnp.int32) % N_NODES
    pad_dst = N_NODES + jnp.arange(npad_e, dtype=jnp.int32) % NTRASH
    src2d = jnp.concatenate([src, pad_src]).reshape(-1, BLK)
    dst2d = jnp.concatenate([dst, pad_dst]).reshape(-1, BLK)

    deg0, deg1 = _sc_deg(dst2d)
    h0, g0, g1, dinv = _tc_fc1(x, fc1_w, fc1_b.reshape(1, -1), deg0, deg1)
    hn = h0
    for i in range(NUM_LAYERS):
        beta = log(THETA / (i + 1) + 1.0)
        ag0, ag1 = _sc_msg(g0, g1, src2d, dst2d)
        hn, g0, g1 = _tc_layer(beta, ag0, ag1, g0, g1, h0, dinv, conv_ws[i])
    return _tc_fc2(hn, fc2_w, fc2_b.reshape(1, -1))


# speedup vs baseline: 1.0322x; 1.0322x over previous
import functools
from math import log

import jax
import jax.numpy as jnp
from jax import lax
from jax.experimental import pallas as pl
from jax.experimental.pallas import tpu as pltpu
from jax.experimental.pallas import tpu_sc as plsc

N_NODES = 10000
N_EDGES = 320000
DIM_NODE = 128
DIM_HIDDEN = 256
HALF = DIM_HIDDEN // 2
NUM_CLASSES = 40
ALPHA = 0.1
THETA = 0.5
NUM_LAYERS = 8

NC = 2
NS = 16
NPAD = 10240
ROWS_PER_TILE = NPAD // NS
ZROWS = 128
DEGW = 16

BLK = 128
BATCH = 8
EPAD = 327680
NBLK_DEG = EPAD // (NC * NS * BLK)
NBATCH_DEG = NBLK_DEG // BATCH
BLKM = 120
EPADM = 322560
NBLK_TILE = EPADM // (NS * BLKM)
NBATCH = NBLK_TILE // BATCH
NTRASH = 240

_MESH = plsc.VectorSubcoreMesh(core_axis_name="c", subcore_axis_name="s")


def _zero_fill(ref, nrows, width):
    def row(i, _):
        for j in range(width // 16):
            ref[i, pl.ds(j * 16, 16)] = jnp.zeros((16,), jnp.float32)
        return 0
    lax.fori_loop(0, nrows, row, 0)


def _zero_acc(zbuf_v, acc_sh, s, width):
    _zero_fill(zbuf_v, ZROWS, width)
    for k in range(ROWS_PER_TILE // ZROWS):
        pltpu.sync_copy(zbuf_v, acc_sh.at[pl.ds(s * ROWS_PER_TILE + k * ZROWS, ZROWS)])


def _deg_body(dst2d_hbm, deg0_hbm, deg1_hbm, ones_v, zbuf_v, idx_v, acc_sh):
    c = lax.axis_index("c")
    s = lax.axis_index("s")
    wid = s * NC + c

    def fill_ones(i, _):
        ones_v[i, :] = jnp.ones((16,), jnp.float32)
        return 0
    lax.fori_loop(0, BLK, fill_ones, 0)

    _zero_acc(zbuf_v, acc_sh, s, DEGW)
    plsc.subcore_barrier()

    blk0 = wid * NBLK_DEG
    def batch(bi, _):
        pltpu.sync_copy(dst2d_hbm.at[pl.ds(blk0 + bi * BATCH, BATCH)], idx_v)
        for j in range(BATCH):
            pltpu.sync_copy(ones_v, acc_sh.at[idx_v.at[j]], add=True)
        return 0
    lax.fori_loop(0, NBATCH_DEG, batch, 0)
    plsc.subcore_barrier()

    rows = pl.ds(s * ROWS_PER_TILE, ROWS_PER_TILE)
    @pl.when(c == 0)
    def _():
        pltpu.sync_copy(acc_sh.at[rows], deg0_hbm.at[rows])
    @pl.when(c == 1)
    def _():
        pltpu.sync_copy(acc_sh.at[rows], deg1_hbm.at[rows])


_sc_deg = functools.partial(
    pl.kernel,
    out_type=(
        jax.ShapeDtypeStruct((NPAD, DEGW), jnp.float32),
        jax.ShapeDtypeStruct((NPAD, DEGW), jnp.float32),
    ),
    mesh=_MESH,
    scratch_types=[
        pltpu.VMEM((BLK, DEGW), jnp.float32),
        pltpu.VMEM((ZROWS, DEGW), jnp.float32),
        pltpu.VMEM((BATCH, BLK), jnp.int32),
        pltpu.VMEM_SHARED((NPAD, DEGW), jnp.float32),
    ],
)(_deg_body)


def _msg_body(g0_hbm, g1_hbm, src2d_hbm, dst2d_hbm, agg0_hbm, agg1_hbm,
              idxs_v, idxd_v, rows0_v, rows1_v, rows2_v, acc_sh,
              semg0, semg1, semg2):
    c = lax.axis_index("c")
    s = lax.axis_index("s")

    _zero_fill(rows0_v, BLKM, HALF)
    base = s * ROWS_PER_TILE
    for k in range(5):
        pltpu.sync_copy(rows0_v, acc_sh.at[pl.ds(base + k * BLKM, BLKM)])
    pltpu.sync_copy(rows0_v.at[pl.ds(0, 40)], acc_sh.at[pl.ds(base + 600, 40)])
    plsc.subcore_barrier()

    blk0 = s * NBLK_TILE

    def run(g_hbm):
        def batch(bi, _):
            row0 = blk0 + bi * BATCH
            pltpu.sync_copy(src2d_hbm.at[pl.ds(row0, BATCH)], idxs_v)
            pltpu.sync_copy(dst2d_hbm.at[pl.ds(row0, BATCH)], idxd_v)
            bufs = [rows0_v, rows1_v, rows2_v]
            sems = [semg0, semg1, semg2]
            d = [None] * BATCH
            d[0] = pltpu.async_copy(g_hbm.at[idxs_v.at[0]], bufs[0], sems[0])
            d[1] = pltpu.async_copy(g_hbm.at[idxs_v.at[1]], bufs[1], sems[1])
            for j in range(2, BATCH):
                d[j] = pltpu.async_copy(
                    g_hbm.at[idxs_v.at[j]], bufs[j % 3], sems[j % 3])
                d[j - 2].wait()
                pltpu.sync_copy(bufs[(j - 2) % 3],
                                acc_sh.at[idxd_v.at[j - 2]], add=True)
            for j in range(BATCH - 2, BATCH):
                d[j].wait()
                pltpu.sync_copy(bufs[j % 3],
                                acc_sh.at[idxd_v.at[j]], add=True)
            return 0
        lax.fori_loop(0, NBATCH, batch, 0)

    @pl.when(c == 0)
    def _():
        run(g0_hbm)
    @pl.when(c == 1)
    def _():
        run(g1_hbm)
    plsc.subcore_barrier()

    rows = pl.ds(s * ROWS_PER_TILE, ROWS_PER_TILE)
    @pl.when(c == 0)
    def _():
        pltpu.sync_copy(acc_sh.at[rows], agg0_hbm.at[rows])
    @pl.when(c == 1)
    def _():
        pltpu.sync_copy(acc_sh.at[rows], agg1_hbm.at[rows])


_sc_msg = functools.partial(
    pl.kernel,
    out_type=(
        jax.ShapeDtypeStruct((NPAD, HALF), jnp.float32),
        jax.ShapeDtypeStruct((NPAD, HALF), jnp.float32),
    ),
    mesh=_MESH,
    scratch_types=[
        pltpu.VMEM((BATCH, BLKM), jnp.int32),
        pltpu.VMEM((BATCH, BLKM), jnp.int32),
        pltpu.VMEM((BLKM, HALF), jnp.float32),
        pltpu.VMEM((BLKM, HALF), jnp.float32),
        pltpu.VMEM((BLKM, HALF), jnp.float32),
        pltpu.VMEM_SHARED((NPAD, HALF), jnp.float32),
        pltpu.SemaphoreType.DMA,
        pltpu.SemaphoreType.DMA,
        pltpu.SemaphoreType.DMA,
    ],
)(_msg_body)


_RB = 1000
_GRID = N_NODES // _RB


def _fc1_kernel(x_ref, w_ref, b_ref, d0_ref, d1_ref,
                h0_ref, g0_ref, g1_ref, dinv_ref):
    h = jnp.maximum(
        jnp.dot(x_ref[...], w_ref[...], preferred_element_type=jnp.float32)
        + b_ref[...], 0.0)
    deg = 1.0 + d0_ref[:, 0:1] + d1_ref[:, 0:1]
    dinv = lax.rsqrt(deg)
    h0_ref[...] = h
    g0_ref[...] = dinv * h[:, :HALF]
    g1_ref[...] = dinv * h[:, HALF:]
    dinv_ref[...] = jnp.broadcast_to(dinv, (_RB, HALF))


def _tc_fc1(x, fc1_w, fc1_b, deg0, deg1):
    return pl.pallas_call(
        _fc1_kernel,
        grid=(_GRID,),
        in_specs=[
            pl.BlockSpec((_RB, DIM_NODE), lambda b: (b, 0)),
            pl.BlockSpec((DIM_NODE, DIM_HIDDEN), lambda b: (0, 0)),
            pl.BlockSpec((1, DIM_HIDDEN), lambda b: (0, 0)),
            pl.BlockSpec((_RB, DEGW), lambda b: (b, 0)),
            pl.BlockSpec((_RB, DEGW), lambda b: (b, 0)),
        ],
        out_specs=[
            pl.BlockSpec((_RB, DIM_HIDDEN), lambda b: (b, 0)),
            pl.BlockSpec((_RB, HALF), lambda b: (b, 0)),
            pl.BlockSpec((_RB, HALF), lambda b: (b, 0)),
            pl.BlockSpec((_RB, HALF), lambda b: (b, 0)),
        ],
        out_shape=[
            jax.ShapeDtypeStruct((N_NODES, DIM_HIDDEN), jnp.float32),
            jax.ShapeDtypeStruct((N_NODES, HALF), jnp.float32),
            jax.ShapeDtypeStruct((N_NODES, HALF), jnp.float32),
            jax.ShapeDtypeStruct((N_NODES, HALF), jnp.float32),
        ],
    )(x, fc1_w, fc1_b, deg0, deg1)


def _layer_kernel(beta, ag0_ref, ag1_ref, g0_ref, g1_ref, h0_ref, dinv_ref,
                  w_ref, hn_ref, g0n_ref, g1n_ref):
    dinv = dinv_ref[...]
    a0 = dinv * (ag0_ref[...] + g0_ref[...])
    a1 = dinv * (ag1_ref[...] + g1_ref[...])
    z = (1.0 - ALPHA) * jnp.concatenate([a0, a1], axis=1) + ALPHA * h0_ref[...]
    out = (1.0 - beta) * z + beta * jnp.dot(
        z, w_ref[...], preferred_element_type=jnp.float32)
    h = jnp.maximum(out, 0.0)
    hn_ref[...] = h
    g0n_ref[...] = dinv * h[:, :HALF]
    g1n_ref[...] = dinv * h[:, HALF:]


def _tc_layer(beta, ag0, ag1, g0, g1, h0, dinv, w):
    return pl.pallas_call(
        functools.partial(_layer_kernel, beta),
        grid=(_GRID,),
        in_specs=[
            pl.BlockSpec((_RB, HALF), lambda b: (b, 0)),
            pl.BlockSpec((_RB, HALF), lambda b: (b, 0)),
            pl.BlockSpec((_RB, HALF), lambda b: (b, 0)),
            pl.BlockSpec((_RB, HALF), lambda b: (b, 0)),
            pl.BlockSpec((_RB, DIM_HIDDEN), lambda b: (b, 0)),
            pl.BlockSpec((_RB, HALF), lambda b: (b, 0)),
            pl.BlockSpec((DIM_HIDDEN, DIM_HIDDEN), lambda b: (0, 0)),
        ],
        out_specs=[
            pl.BlockSpec((_RB, DIM_HIDDEN), lambda b: (b, 0)),
            pl.BlockSpec((_RB, HALF), lambda b: (b, 0)),
            pl.BlockSpec((_RB, HALF), lambda b: (b, 0)),
        ],
        out_shape=[
            jax.ShapeDtypeStruct((N_NODES, DIM_HIDDEN), jnp.float32),
            jax.ShapeDtypeStruct((N_NODES, HALF), jnp.float32),
            jax.ShapeDtypeStruct((N_NODES, HALF), jnp.float32),
        ],
    )(ag0, ag1, g0, g1, h0, dinv, w)


def _fc2_kernel(h_ref, w_ref, b_ref, o_ref):
    o_ref[...] = jnp.dot(h_ref[...], w_ref[...],
                         preferred_element_type=jnp.float32) + b_ref[...]


def _tc_fc2(h, fc2_w, fc2_b):
    return pl.pallas_call(
        _fc2_kernel,
        grid=(_GRID,),
        in_specs=[
            pl.BlockSpec((_RB, DIM_HIDDEN), lambda b: (b, 0)),
            pl.BlockSpec((DIM_HIDDEN, NUM_CLASSES), lambda b: (0, 0)),
            pl.BlockSpec((1, NUM_CLASSES), lambda b: (0, 0)),
        ],
        out_specs=pl.BlockSpec((_RB, NUM_CLASSES), lambda b: (b, 0)),
        out_shape=jax.ShapeDtypeStruct((N_NODES, NUM_CLASSES), jnp.float32),
    )(h, fc2_w, fc2_b)


def kernel(x, edge_index, fc1_w, fc1_b, conv_ws, fc2_w, fc2_b):
    src = edge_index[0]
    dst = edge_index[1]
    npad_deg = EPAD - N_EDGES
    dst2d_deg = jnp.concatenate(
        [dst, N_NODES + jnp.arange(npad_deg, dtype=jnp.int32) % NTRASH]
    ).reshape(-1, BLK)
    npad_m = EPADM - N_EDGES
    src2d = jnp.concatenate(
        [src, jnp.arange(npad_m, dtype=jnp.int32) % N_NODES]).reshape(-1, BLKM)
    dst2d = jnp.concatenate(
        [dst, N_NODES + jnp.arange(npad_m, dtype=jnp.int32) % NTRASH]
    ).reshape(-1, BLKM)

    deg0, deg1 = _sc_deg(dst2d_deg)
    h0, g0, g1, dinv = _tc_fc1(x, fc1_w, fc1_b.reshape(1, -1), deg0, deg1)
    hn = h0
    for i in range(NUM_LAYERS):
        beta = log(THETA / (i + 1) + 1.0)
        ag0, ag1 = _sc_msg(g0, g1, src2d, dst2d)
        hn, g0, g1 = _tc_layer(beta, ag0, ag1, g0, g1, h0, dinv, conv_ws[i])
    return _tc_fc2(hn, fc2_w, fc2_b.reshape(1, -1))


# speedup vs baseline: 1.0340x; 1.0017x over previous
import functools
from math import log

import jax
import jax.numpy as jnp
from jax import lax
from jax.experimental import pallas as pl
from jax.experimental.pallas import tpu as pltpu
from jax.experimental.pallas import tpu_sc as plsc

N_NODES = 10000
N_EDGES = 320000
DIM_NODE = 128
DIM_HIDDEN = 256
HALF = DIM_HIDDEN // 2
NUM_CLASSES = 40
ALPHA = 0.1
THETA = 0.5
NUM_LAYERS = 8

NC = 2
NS = 16
NPAD = 10240
ROWS_PER_TILE = NPAD // NS
ZROWS = 128
DEGW = 16

BLK = 128
BATCH = 8
EPAD = 327680
NBLK_DEG = EPAD // (NC * NS * BLK)
NBATCH_DEG = NBLK_DEG // BATCH
BLKM = 120
EPADM = 322560
NBLK_TILE = EPADM // (NS * BLKM)
NBATCH = NBLK_TILE // BATCH
NTRASH = 240

_MESH = plsc.VectorSubcoreMesh(core_axis_name="c", subcore_axis_name="s")


def _zero_fill(ref, nrows, width):
    def row(i, _):
        for j in range(width // 16):
            ref[i, pl.ds(j * 16, 16)] = jnp.zeros((16,), jnp.float32)
        return 0
    lax.fori_loop(0, nrows, row, 0)


def _zero_acc(zbuf_v, acc_sh, s, width):
    _zero_fill(zbuf_v, ZROWS, width)
    for k in range(ROWS_PER_TILE // ZROWS):
        pltpu.sync_copy(zbuf_v, acc_sh.at[pl.ds(s * ROWS_PER_TILE + k * ZROWS, ZROWS)])


def _deg_body(dst2d_hbm, deg0_hbm, deg1_hbm, ones_v, zbuf_v, idx_v, acc_sh):
    c = lax.axis_index("c")
    s = lax.axis_index("s")
    wid = s * NC + c

    def fill_ones(i, _):
        ones_v[i, :] = jnp.ones((16,), jnp.float32)
        return 0
    lax.fori_loop(0, BLK, fill_ones, 0)

    _zero_acc(zbuf_v, acc_sh, s, DEGW)
    plsc.subcore_barrier()

    blk0 = wid * NBLK_DEG
    def batch(bi, _):
        pltpu.sync_copy(dst2d_hbm.at[pl.ds(blk0 + bi * BATCH, BATCH)], idx_v)
        for j in range(BATCH):
            pltpu.sync_copy(ones_v, acc_sh.at[idx_v.at[j]], add=True)
        return 0
    lax.fori_loop(0, NBATCH_DEG, batch, 0)
    plsc.subcore_barrier()

    rows = pl.ds(s * ROWS_PER_TILE, ROWS_PER_TILE)
    @pl.when(c == 0)
    def _():
        pltpu.sync_copy(acc_sh.at[rows], deg0_hbm.at[rows])
    @pl.when(c == 1)
    def _():
        pltpu.sync_copy(acc_sh.at[rows], deg1_hbm.at[rows])


_sc_deg = functools.partial(
    pl.kernel,
    out_type=(
        jax.ShapeDtypeStruct((NPAD, DEGW), jnp.float32),
        jax.ShapeDtypeStruct((NPAD, DEGW), jnp.float32),
    ),
    mesh=_MESH,
    scratch_types=[
        pltpu.VMEM((BLK, DEGW), jnp.float32),
        pltpu.VMEM((ZROWS, DEGW), jnp.float32),
        pltpu.VMEM((BATCH, BLK), jnp.int32),
        pltpu.VMEM_SHARED((NPAD, DEGW), jnp.float32),
    ],
)(_deg_body)


def _msg_body(g0_hbm, g1_hbm, src2d_hbm, dst2d_hbm, agg0_hbm, agg1_hbm,
              idxs_v, idxd_v, rows0_v, rows1_v, rows2_v, acc_sh,
              semg0, semg1, semg2, sems0, sems1, sems2):
    c = lax.axis_index("c")
    s = lax.axis_index("s")

    _zero_fill(rows0_v, BLKM, HALF)
    base = s * ROWS_PER_TILE
    for k in range(5):
        pltpu.sync_copy(rows0_v, acc_sh.at[pl.ds(base + k * BLKM, BLKM)])
    pltpu.sync_copy(rows0_v.at[pl.ds(0, 40)], acc_sh.at[pl.ds(base + 600, 40)])
    plsc.subcore_barrier()

    blk0 = s * NBLK_TILE

    def run(g_hbm):
        def batch(bi, _):
            row0 = blk0 + bi * BATCH
            pltpu.sync_copy(src2d_hbm.at[pl.ds(row0, BATCH)], idxs_v)
            pltpu.sync_copy(dst2d_hbm.at[pl.ds(row0, BATCH)], idxd_v)
            bufs = [rows0_v, rows1_v, rows2_v]
            gsem = [semg0, semg1, semg2]
            ssem = [sems0, sems1, sems2]
            d = [None] * BATCH
            sd = [None] * BATCH
            d[0] = pltpu.async_copy(g_hbm.at[idxs_v.at[0]], bufs[0], gsem[0])
            d[1] = pltpu.async_copy(g_hbm.at[idxs_v.at[1]], bufs[1], gsem[1])
            for j in range(2, BATCH):
                if j >= 3:
                    sd[j - 3].wait()
                d[j] = pltpu.async_copy(
                    g_hbm.at[idxs_v.at[j]], bufs[j % 3], gsem[j % 3])
                d[j - 2].wait()
                sd[j - 2] = pltpu.async_copy(
                    bufs[(j - 2) % 3], acc_sh.at[idxd_v.at[j - 2]],
                    ssem[(j - 2) % 3], add=True)
            for j in range(BATCH - 2, BATCH):
                d[j].wait()
                sd[j] = pltpu.async_copy(
                    bufs[j % 3], acc_sh.at[idxd_v.at[j]], ssem[j % 3],
                    add=True)
            for j in range(BATCH - 3, BATCH):
                sd[j].wait()
            return 0
        lax.fori_loop(0, NBATCH, batch, 0)

    @pl.when(c == 0)
    def _():
        run(g0_hbm)
    @pl.when(c == 1)
    def _():
        run(g1_hbm)
    plsc.subcore_barrier()

    rows = pl.ds(s * ROWS_PER_TILE, ROWS_PER_TILE)
    @pl.when(c == 0)
    def _():
        pltpu.sync_copy(acc_sh.at[rows], agg0_hbm.at[rows])
    @pl.when(c == 1)
    def _():
        pltpu.sync_copy(acc_sh.at[rows], agg1_hbm.at[rows])


_sc_msg = functools.partial(
    pl.kernel,
    out_type=(
        jax.ShapeDtypeStruct((NPAD, HALF), jnp.float32),
        jax.ShapeDtypeStruct((NPAD, HALF), jnp.float32),
    ),
    mesh=_MESH,
    scratch_types=[
        pltpu.VMEM((BATCH, BLKM), jnp.int32),
        pltpu.VMEM((BATCH, BLKM), jnp.int32),
        pltpu.VMEM((BLKM, HALF), jnp.float32),
        pltpu.VMEM((BLKM, HALF), jnp.float32),
        pltpu.VMEM((BLKM, HALF), jnp.float32),
        pltpu.VMEM_SHARED((NPAD, HALF), jnp.float32),
        pltpu.SemaphoreType.DMA,
        pltpu.SemaphoreType.DMA,
        pltpu.SemaphoreType.DMA,
        pltpu.SemaphoreType.DMA,
        pltpu.SemaphoreType.DMA,
        pltpu.SemaphoreType.DMA,
    ],
)(_msg_body)


_RB = 1000
_GRID = N_NODES // _RB


def _fc1_kernel(x_ref, w_ref, b_ref, d0_ref, d1_ref,
                h0_ref, g0_ref, g1_ref, dinv_ref):
    h = jnp.maximum(
        jnp.dot(x_ref[...], w_ref[...], preferred_element_type=jnp.float32)
        + b_ref[...], 0.0)
    deg = 1.0 + d0_ref[:, 0:1] + d1_ref[:, 0:1]
    dinv = lax.rsqrt(deg)
    h0_ref[...] = h
    g0_ref[...] = dinv * h[:, :HALF]
    g1_ref[...] = dinv * h[:, HALF:]
    dinv_ref[...] = jnp.broadcast_to(dinv, (_RB, HALF))


def _tc_fc1(x, fc1_w, fc1_b, deg0, deg1):
    return pl.pallas_call(
        _fc1_kernel,
        grid=(_GRID,),
        in_specs=[
            pl.BlockSpec((_RB, DIM_NODE), lambda b: (b, 0)),
            pl.BlockSpec((DIM_NODE, DIM_HIDDEN), lambda b: (0, 0)),
            pl.BlockSpec((1, DIM_HIDDEN), lambda b: (0, 0)),
            pl.BlockSpec((_RB, DEGW), lambda b: (b, 0)),
            pl.BlockSpec((_RB, DEGW), lambda b: (b, 0)),
        ],
        out_specs=[
            pl.BlockSpec((_RB, DIM_HIDDEN), lambda b: (b, 0)),
            pl.BlockSpec((_RB, HALF), lambda b: (b, 0)),
            pl.BlockSpec((_RB, HALF), lambda b: (b, 0)),
            pl.BlockSpec((_RB, HALF), lambda b: (b, 0)),
        ],
        out_shape=[
            jax.ShapeDtypeStruct((N_NODES, DIM_HIDDEN), jnp.float32),
            jax.ShapeDtypeStruct((N_NODES, HALF), jnp.float32),
            jax.ShapeDtypeStruct((N_NODES, HALF), jnp.float32),
            jax.ShapeDtypeStruct((N_NODES, HALF), jnp.float32),
        ],
    )(x, fc1_w, fc1_b, deg0, deg1)


def _layer_kernel(beta, ag0_ref, ag1_ref, g0_ref, g1_ref, h0_ref, dinv_ref,
                  w_ref, hn_ref, g0n_ref, g1n_ref):
    dinv = dinv_ref[...]
    a0 = dinv * (ag0_ref[...] + g0_ref[...])
    a1 = dinv * (ag1_ref[...] + g1_ref[...])
    z = (1.0 - ALPHA) * jnp.concatenate([a0, a1], axis=1) + ALPHA * h0_ref[...]
    out = (1.0 - beta) * z + beta * jnp.dot(
        z, w_ref[...], preferred_element_type=jnp.float32)
    h = jnp.maximum(out, 0.0)
    hn_ref[...] = h
    g0n_ref[...] = dinv * h[:, :HALF]
    g1n_ref[...] = dinv * h[:, HALF:]


def _tc_layer(beta, ag0, ag1, g0, g1, h0, dinv, w):
    return pl.pallas_call(
        functools.partial(_layer_kernel, beta),
        grid=(_GRID,),
        in_specs=[
            pl.BlockSpec((_RB, HALF), lambda b: (b, 0)),
            pl.BlockSpec((_RB, HALF), lambda b: (b, 0)),
            pl.BlockSpec((_RB, HALF), lambda b: (b, 0)),
            pl.BlockSpec((_RB, HALF), lambda b: (b, 0)),
            pl.BlockSpec((_RB, DIM_HIDDEN), lambda b: (b, 0)),
            pl.BlockSpec((_RB, HALF), lambda b: (b, 0)),
            pl.BlockSpec((DIM_HIDDEN, DIM_HIDDEN), lambda b: (0, 0)),
        ],
        out_specs=[
            pl.BlockSpec((_RB, DIM_HIDDEN), lambda b: (b, 0)),
            pl.BlockSpec((_RB, HALF), lambda b: (b, 0)),
            pl.BlockSpec((_RB, HALF), lambda b: (b, 0)),
        ],
        out_shape=[
            jax.ShapeDtypeStruct((N_NODES, DIM_HIDDEN), jnp.float32),
            jax.ShapeDtypeStruct((N_NODES, HALF), jnp.float32),
            jax.ShapeDtypeStruct((N_NODES, HALF), jnp.float32),
        ],
    )(ag0, ag1, g0, g1, h0, dinv, w)


def _fc2_kernel(h_ref, w_ref, b_ref, o_ref):
    o_ref[...] = jnp.dot(h_ref[...], w_ref[...],
                         preferred_element_type=jnp.float32) + b_ref[...]


def _tc_fc2(h, fc2_w, fc2_b):
    return pl.pallas_call(
        _fc2_kernel,
        grid=(_GRID,),
        in_specs=[
            pl.BlockSpec((_RB, DIM_HIDDEN), lambda b: (b, 0)),
            pl.BlockSpec((DIM_HIDDEN, NUM_CLASSES), lambda b: (0, 0)),
            pl.BlockSpec((1, NUM_CLASSES), lambda b: (0, 0)),
        ],
        out_specs=pl.BlockSpec((_RB, NUM_CLASSES), lambda b: (b, 0)),
        out_shape=jax.ShapeDtypeStruct((N_NODES, NUM_CLASSES), jnp.float32),
    )(h, fc2_w, fc2_b)


def kernel(x, edge_index, fc1_w, fc1_b, conv_ws, fc2_w, fc2_b):
    src = edge_index[0]
    dst = edge_index[1]
    npad_deg = EPAD - N_EDGES
    dst2d_deg = jnp.concatenate(
        [dst, N_NODES + jnp.arange(npad_deg, dtype=jnp.int32) % NTRASH]
    ).reshape(-1, BLK)
    npad_m = EPADM - N_EDGES
    src2d = jnp.concatenate(
        [src, jnp.arange(npad_m, dtype=jnp.int32) % N_NODES]).reshape(-1, BLKM)
    dst2d = jnp.concatenate(
        [dst, N_NODES + jnp.arange(npad_m, dtype=jnp.int32) % NTRASH]
    ).reshape(-1, BLKM)

    deg0, deg1 = _sc_deg(dst2d_deg)
    h0, g0, g1, dinv = _tc_fc1(x, fc1_w, fc1_b.reshape(1, -1), deg0, deg1)
    hn = h0
    for i in range(NUM_LAYERS):
        beta = log(THETA / (i + 1) + 1.0)
        ag0, ag1 = _sc_msg(g0, g1, src2d, dst2d)
        hn, g0, g1 = _tc_layer(beta, ag0, ag1, g0, g1, h0, dinv, conv_ws[i])
    return _tc_fc2(hn, fc2_w, fc2_b.reshape(1, -1))


# speedup vs baseline: 1.1763x; 1.1376x over previous
import functools
from math import log

import jax
import jax.numpy as jnp
from jax import lax
from jax.experimental import pallas as pl
from jax.experimental.pallas import tpu as pltpu
from jax.experimental.pallas import tpu_sc as plsc

N_NODES = 10000
N_EDGES = 320000
DIM_NODE = 128
DIM_HIDDEN = 256
HALF = DIM_HIDDEN // 2
NUM_CLASSES = 40
ALPHA = 0.1
THETA = 0.5
NUM_LAYERS = 8

NC = 2
NS = 16
NPAD = 10240
ROWS_PER_TILE = NPAD // NS
ZROWS = 128
DEGW = 16

BLK = 128
BATCH = 8
EPAD = 327680
NBLK_DEG = EPAD // (NC * NS * BLK)
NBATCH_DEG = NBLK_DEG // BATCH
BLKM = 120
EPADM = 322560
NBLK_TILE = EPADM // (NS * BLKM)
BATCHM = 56
NBATCH = NBLK_TILE // BATCHM
NTRASH = 240

_MESH = plsc.VectorSubcoreMesh(core_axis_name="c", subcore_axis_name="s")


def _zero_fill(ref, nrows, width):
    def row(i, _):
        for j in range(width // 16):
            ref[i, pl.ds(j * 16, 16)] = jnp.zeros((16,), jnp.float32)
        return 0
    lax.fori_loop(0, nrows, row, 0)


def _zero_acc(zbuf_v, acc_sh, s, width):
    _zero_fill(zbuf_v, ZROWS, width)
    for k in range(ROWS_PER_TILE // ZROWS):
        pltpu.sync_copy(zbuf_v, acc_sh.at[pl.ds(s * ROWS_PER_TILE + k * ZROWS, ZROWS)])


def _deg_body(dst2d_hbm, deg0_hbm, deg1_hbm, ones_v, zbuf_v, idx_v, acc_sh):
    c = lax.axis_index("c")
    s = lax.axis_index("s")
    wid = s * NC + c

    def fill_ones(i, _):
        ones_v[i, :] = jnp.ones((16,), jnp.float32)
        return 0
    lax.fori_loop(0, BLK, fill_ones, 0)

    _zero_acc(zbuf_v, acc_sh, s, DEGW)
    plsc.subcore_barrier()

    blk0 = wid * NBLK_DEG
    def batch(bi, _):
        pltpu.sync_copy(dst2d_hbm.at[pl.ds(blk0 + bi * BATCH, BATCH)], idx_v)
        for j in range(BATCH):
            pltpu.sync_copy(ones_v, acc_sh.at[idx_v.at[j]], add=True)
        return 0
    lax.fori_loop(0, NBATCH_DEG, batch, 0)
    plsc.subcore_barrier()

    rows = pl.ds(s * ROWS_PER_TILE, ROWS_PER_TILE)
    @pl.when(c == 0)
    def _():
        pltpu.sync_copy(acc_sh.at[rows], deg0_hbm.at[rows])
    @pl.when(c == 1)
    def _():
        pltpu.sync_copy(acc_sh.at[rows], deg1_hbm.at[rows])


_sc_deg = functools.partial(
    pl.kernel,
    out_type=(
        jax.ShapeDtypeStruct((NPAD, DEGW), jnp.float32),
        jax.ShapeDtypeStruct((NPAD, DEGW), jnp.float32),
    ),
    mesh=_MESH,
    scratch_types=[
        pltpu.VMEM((BLK, DEGW), jnp.float32),
        pltpu.VMEM((ZROWS, DEGW), jnp.float32),
        pltpu.VMEM((BATCH, BLK), jnp.int32),
        pltpu.VMEM_SHARED((NPAD, DEGW), jnp.float32),
    ],
)(_deg_body)


def _msg_body(g0_hbm, g1_hbm, src2d_hbm, dst2d_hbm, agg0_hbm, agg1_hbm,
              idxs_v, idxd_v, rows0_v, rows1_v, acc_sh,
              semg0, semg1, sems0, sems1):
    c = lax.axis_index("c")
    s = lax.axis_index("s")

    _zero_fill(rows0_v, BLKM, HALF)
    base = s * ROWS_PER_TILE
    for k in range(5):
        pltpu.sync_copy(rows0_v, acc_sh.at[pl.ds(base + k * BLKM, BLKM)])
    pltpu.sync_copy(rows0_v.at[pl.ds(0, 40)], acc_sh.at[pl.ds(base + 600, 40)])
    plsc.subcore_barrier()

    blk0 = s * NBLK_TILE

    def run(g_hbm):
        def batch(bi, _):
            row0 = blk0 + bi * BATCHM
            pltpu.sync_copy(src2d_hbm.at[pl.ds(row0, BATCHM)], idxs_v)
            pltpu.sync_copy(dst2d_hbm.at[pl.ds(row0, BATCHM)], idxd_v)
            bufs = [rows0_v, rows1_v]
            gsem = [semg0, semg1]
            ssem = [sems0, sems1]
            d = [None] * BATCHM
            sd = [None] * BATCHM
            d[0] = pltpu.async_copy(g_hbm.at[idxs_v.at[0]], bufs[0], gsem[0])
            for j in range(1, BATCHM):
                if j >= 2:
                    sd[j - 2].wait()
                d[j] = pltpu.async_copy(
                    g_hbm.at[idxs_v.at[j]], bufs[j % 2], gsem[j % 2])
                d[j - 1].wait()
                sd[j - 1] = pltpu.async_copy(
                    bufs[(j - 1) % 2], acc_sh.at[idxd_v.at[j - 1]],
                    ssem[(j - 1) % 2], add=True)
            d[BATCHM - 1].wait()
            sd[BATCHM - 1] = pltpu.async_copy(
                bufs[(BATCHM - 1) % 2], acc_sh.at[idxd_v.at[BATCHM - 1]],
                ssem[(BATCHM - 1) % 2], add=True)
            sd[BATCHM - 2].wait()
            sd[BATCHM - 1].wait()
            return 0
        lax.fori_loop(0, NBATCH, batch, 0)

    @pl.when(c == 0)
    def _():
        run(g0_hbm)
    @pl.when(c == 1)
    def _():
        run(g1_hbm)
    plsc.subcore_barrier()

    rows = pl.ds(s * ROWS_PER_TILE, ROWS_PER_TILE)
    @pl.when(c == 0)
    def _():
        pltpu.sync_copy(acc_sh.at[rows], agg0_hbm.at[rows])
    @pl.when(c == 1)
    def _():
        pltpu.sync_copy(acc_sh.at[rows], agg1_hbm.at[rows])


_sc_msg = functools.partial(
    pl.kernel,
    out_type=(
        jax.ShapeDtypeStruct((NPAD, HALF), jnp.float32),
        jax.ShapeDtypeStruct((NPAD, HALF), jnp.float32),
    ),
    mesh=_MESH,
    scratch_types=[
        pltpu.VMEM((BATCHM, BLKM), jnp.int32),
        pltpu.VMEM((BATCHM, BLKM), jnp.int32),
        pltpu.VMEM((BLKM, HALF), jnp.float32),
        pltpu.VMEM((BLKM, HALF), jnp.float32),
        pltpu.VMEM_SHARED((NPAD, HALF), jnp.float32),
        pltpu.SemaphoreType.DMA,
        pltpu.SemaphoreType.DMA,
        pltpu.SemaphoreType.DMA,
        pltpu.SemaphoreType.DMA,
    ],
)(_msg_body)


_RB = 1000
_GRID = N_NODES // _RB


def _fc1_kernel(x_ref, w_ref, b_ref, d0_ref, d1_ref,
                h0_ref, g0_ref, g1_ref, dinv_ref):
    h = jnp.maximum(
        jnp.dot(x_ref[...], w_ref[...], preferred_element_type=jnp.float32)
        + b_ref[...], 0.0)
    deg = 1.0 + d0_ref[:, 0:1] + d1_ref[:, 0:1]
    dinv = lax.rsqrt(deg)
    h0_ref[...] = h
    g0_ref[...] = dinv * h[:, :HALF]
    g1_ref[...] = dinv * h[:, HALF:]
    dinv_ref[...] = jnp.broadcast_to(dinv, (_RB, HALF))


def _tc_fc1(x, fc1_w, fc1_b, deg0, deg1):
    return pl.pallas_call(
        _fc1_kernel,
        grid=(_GRID,),
        in_specs=[
            pl.BlockSpec((_RB, DIM_NODE), lambda b: (b, 0)),
            pl.BlockSpec((DIM_NODE, DIM_HIDDEN), lambda b: (0, 0)),
            pl.BlockSpec((1, DIM_HIDDEN), lambda b: (0, 0)),
            pl.BlockSpec((_RB, DEGW), lambda b: (b, 0)),
            pl.BlockSpec((_RB, DEGW), lambda b: (b, 0)),
        ],
        out_specs=[
            pl.BlockSpec((_RB, DIM_HIDDEN), lambda b: (b, 0)),
            pl.BlockSpec((_RB, HALF), lambda b: (b, 0)),
            pl.BlockSpec((_RB, HALF), lambda b: (b, 0)),
            pl.BlockSpec((_RB, HALF), lambda b: (b, 0)),
        ],
        out_shape=[
            jax.ShapeDtypeStruct((N_NODES, DIM_HIDDEN), jnp.float32),
            jax.ShapeDtypeStruct((N_NODES, HALF), jnp.float32),
            jax.ShapeDtypeStruct((N_NODES, HALF), jnp.float32),
            jax.ShapeDtypeStruct((N_NODES, HALF), jnp.float32),
        ],
    )(x, fc1_w, fc1_b, deg0, deg1)


def _layer_kernel(beta, ag0_ref, ag1_ref, g0_ref, g1_ref, h0_ref, dinv_ref,
                  w_ref, hn_ref, g0n_ref, g1n_ref):
    dinv = dinv_ref[...]
    a0 = dinv * (ag0_ref[...] + g0_ref[...])
    a1 = dinv * (ag1_ref[...] + g1_ref[...])
    z = (1.0 - ALPHA) * jnp.concatenate([a0, a1], axis=1) + ALPHA * h0_ref[...]
    out = (1.0 - beta) * z + beta * jnp.dot(
        z, w_ref[...], preferred_element_type=jnp.float32)
    h = jnp.maximum(out, 0.0)
    hn_ref[...] = h
    g0n_ref[...] = dinv * h[:, :HALF]
    g1n_ref[...] = dinv * h[:, HALF:]


def _tc_layer(beta, ag0, ag1, g0, g1, h0, dinv, w):
    return pl.pallas_call(
        functools.partial(_layer_kernel, beta),
        grid=(_GRID,),
        in_specs=[
            pl.BlockSpec((_RB, HALF), lambda b: (b, 0)),
            pl.BlockSpec((_RB, HALF), lambda b: (b, 0)),
            pl.BlockSpec((_RB, HALF), lambda b: (b, 0)),
            pl.BlockSpec((_RB, HALF), lambda b: (b, 0)),
            pl.BlockSpec((_RB, DIM_HIDDEN), lambda b: (b, 0)),
            pl.BlockSpec((_RB, HALF), lambda b: (b, 0)),
            pl.BlockSpec((DIM_HIDDEN, DIM_HIDDEN), lambda b: (0, 0)),
        ],
        out_specs=[
            pl.BlockSpec((_RB, DIM_HIDDEN), lambda b: (b, 0)),
            pl.BlockSpec((_RB, HALF), lambda b: (b, 0)),
            pl.BlockSpec((_RB, HALF), lambda b: (b, 0)),
        ],
        out_shape=[
            jax.ShapeDtypeStruct((N_NODES, DIM_HIDDEN), jnp.float32),
            jax.ShapeDtypeStruct((N_NODES, HALF), jnp.float32),
            jax.ShapeDtypeStruct((N_NODES, HALF), jnp.float32),
        ],
    )(ag0, ag1, g0, g1, h0, dinv, w)


def _fc2_kernel(h_ref, w_ref, b_ref, o_ref):
    o_ref[...] = jnp.dot(h_ref[...], w_ref[...],
                         preferred_element_type=jnp.float32) + b_ref[...]


def _tc_fc2(h, fc2_w, fc2_b):
    return pl.pallas_call(
        _fc2_kernel,
        grid=(_GRID,),
        in_specs=[
            pl.BlockSpec((_RB, DIM_HIDDEN), lambda b: (b, 0)),
            pl.BlockSpec((DIM_HIDDEN, NUM_CLASSES), lambda b: (0, 0)),
            pl.BlockSpec((1, NUM_CLASSES), lambda b: (0, 0)),
        ],
        out_specs=pl.BlockSpec((_RB, NUM_CLASSES), lambda b: (b, 0)),
        out_shape=jax.ShapeDtypeStruct((N_NODES, NUM_CLASSES), jnp.float32),
    )(h, fc2_w, fc2_b)


def kernel(x, edge_index, fc1_w, fc1_b, conv_ws, fc2_w, fc2_b):
    src = edge_index[0]
    dst = edge_index[1]
    npad_deg = EPAD - N_EDGES
    dst2d_deg = jnp.concatenate(
        [dst, N_NODES + jnp.arange(npad_deg, dtype=jnp.int32) % NTRASH]
    ).reshape(-1, BLK)
    npad_m = EPADM - N_EDGES
    src2d = jnp.concatenate(
        [src, jnp.arange(npad_m, dtype=jnp.int32) % N_NODES]).reshape(-1, BLKM)
    dst2d = jnp.concatenate(
        [dst, N_NODES + jnp.arange(npad_m, dtype=jnp.int32) % NTRASH]
    ).reshape(-1, BLKM)

    deg0, deg1 = _sc_deg(dst2d_deg)
    h0, g0, g1, dinv = _tc_fc1(x, fc1_w, fc1_b.reshape(1, -1), deg0, deg1)
    hn = h0
    for i in range(NUM_LAYERS):
        beta = log(THETA / (i + 1) + 1.0)
        ag0, ag1 = _sc_msg(g0, g1, src2d, dst2d)
        hn, g0, g1 = _tc_layer(beta, ag0, ag1, g0, g1, h0, dinv, conv_ws[i])
    return _tc_fc2(hn, fc2_w, fc2_b.reshape(1, -1))


# speedup vs baseline: 1.1793x; 1.0026x over previous
import functools
from math import log

import jax
import jax.numpy as jnp
from jax import lax
from jax.experimental import pallas as pl
from jax.experimental.pallas import tpu as pltpu
from jax.experimental.pallas import tpu_sc as plsc

N_NODES = 10000
N_EDGES = 320000
DIM_NODE = 128
DIM_HIDDEN = 256
HALF = DIM_HIDDEN // 2
NUM_CLASSES = 40
ALPHA = 0.1
THETA = 0.5
NUM_LAYERS = 8

NC = 2
NS = 16
NPAD = 10240
ROWS_PER_TILE = NPAD // NS
ZROWS = 128
DEGW = 16

BLK = 128
BATCH = 8
EPAD = 327680
NBLK_DEG = EPAD // (NC * NS * BLK)
BATCH_DEG = 40
NBATCH_DEG = NBLK_DEG // BATCH_DEG
BLKM = 120
EPADM = 322560
NBLK_TILE = EPADM // (NS * BLKM)
BATCHM = 56
NBATCH = NBLK_TILE // BATCHM
NTRASH = 240

_MESH = plsc.VectorSubcoreMesh(core_axis_name="c", subcore_axis_name="s")


def _zero_fill(ref, nrows, width):
    def row(i, _):
        for j in range(width // 16):
            ref[i, pl.ds(j * 16, 16)] = jnp.zeros((16,), jnp.float32)
        return 0
    lax.fori_loop(0, nrows, row, 0)


def _zero_acc(zbuf_v, acc_sh, s, width):
    _zero_fill(zbuf_v, ZROWS, width)
    for k in range(ROWS_PER_TILE // ZROWS):
        pltpu.sync_copy(zbuf_v, acc_sh.at[pl.ds(s * ROWS_PER_TILE + k * ZROWS, ZROWS)])


def _deg_body(dst2d_hbm, deg0_hbm, deg1_hbm, ones_v, zbuf_v, idx_v, acc_sh,
              semd):
    c = lax.axis_index("c")
    s = lax.axis_index("s")
    wid = s * NC + c

    def fill_ones(i, _):
        ones_v[i, :] = jnp.ones((16,), jnp.float32)
        return 0
    lax.fori_loop(0, BLK, fill_ones, 0)

    _zero_acc(zbuf_v, acc_sh, s, DEGW)
    plsc.subcore_barrier()

    blk0 = wid * NBLK_DEG
    def batch(bi, _):
        pltpu.sync_copy(
            dst2d_hbm.at[pl.ds(blk0 + bi * BATCH_DEG, BATCH_DEG)], idx_v)
        sd = [pltpu.async_copy(ones_v, acc_sh.at[idx_v.at[j]], semd,
                               add=True)
              for j in range(BATCH_DEG)]
        for dd in sd:
            dd.wait()
        return 0
    lax.fori_loop(0, NBATCH_DEG, batch, 0)
    plsc.subcore_barrier()

    rows = pl.ds(s * ROWS_PER_TILE, ROWS_PER_TILE)
    @pl.when(c == 0)
    def _():
        pltpu.sync_copy(acc_sh.at[rows], deg0_hbm.at[rows])
    @pl.when(c == 1)
    def _():
        pltpu.sync_copy(acc_sh.at[rows], deg1_hbm.at[rows])


_sc_deg = functools.partial(
    pl.kernel,
    out_type=(
        jax.ShapeDtypeStruct((NPAD, DEGW), jnp.float32),
        jax.ShapeDtypeStruct((NPAD, DEGW), jnp.float32),
    ),
    mesh=_MESH,
    scratch_types=[
        pltpu.VMEM((BLK, DEGW), jnp.float32),
        pltpu.VMEM((ZROWS, DEGW), jnp.float32),
        pltpu.VMEM((BATCH_DEG, BLK), jnp.int32),
        pltpu.VMEM_SHARED((NPAD, DEGW), jnp.float32),
        pltpu.SemaphoreType.DMA,
    ],
)(_deg_body)


def _msg_body(g0_hbm, g1_hbm, src2d_hbm, dst2d_hbm, agg0_hbm, agg1_hbm,
              idxs_v, idxd_v, rows0_v, rows1_v, acc_sh,
              semg0, semg1, sems0, sems1):
    c = lax.axis_index("c")
    s = lax.axis_index("s")

    _zero_fill(rows0_v, BLKM, HALF)
    base = s * ROWS_PER_TILE
    for k in range(5):
        pltpu.sync_copy(rows0_v, acc_sh.at[pl.ds(base + k * BLKM, BLKM)])
    pltpu.sync_copy(rows0_v.at[pl.ds(0, 40)], acc_sh.at[pl.ds(base + 600, 40)])
    plsc.subcore_barrier()

    blk0 = s * NBLK_TILE

    def run(g_hbm):
        def batch(bi, _):
            row0 = blk0 + bi * BATCHM
            pltpu.sync_copy(src2d_hbm.at[pl.ds(row0, BATCHM)], idxs_v)
            pltpu.sync_copy(dst2d_hbm.at[pl.ds(row0, BATCHM)], idxd_v)
            bufs = [rows0_v, rows1_v]
            gsem = [semg0, semg1]
            ssem = [sems0, sems1]
            d = [None] * BATCHM
            sd = [None] * BATCHM
            d[0] = pltpu.async_copy(g_hbm.at[idxs_v.at[0]], bufs[0], gsem[0])
            for j in range(1, BATCHM):
                if j >= 2:
                    sd[j - 2].wait()
                d[j] = pltpu.async_copy(
                    g_hbm.at[idxs_v.at[j]], bufs[j % 2], gsem[j % 2])
                d[j - 1].wait()
                sd[j - 1] = pltpu.async_copy(
                    bufs[(j - 1) % 2], acc_sh.at[idxd_v.at[j - 1]],
                    ssem[(j - 1) % 2], add=True)
            d[BATCHM - 1].wait()
            sd[BATCHM - 1] = pltpu.async_copy(
                bufs[(BATCHM - 1) % 2], acc_sh.at[idxd_v.at[BATCHM - 1]],
                ssem[(BATCHM - 1) % 2], add=True)
            sd[BATCHM - 2].wait()
            sd[BATCHM - 1].wait()
            return 0
        lax.fori_loop(0, NBATCH, batch, 0)

    @pl.when(c == 0)
    def _():
        run(g0_hbm)
    @pl.when(c == 1)
    def _():
        run(g1_hbm)
    plsc.subcore_barrier()

    rows = pl.ds(s * ROWS_PER_TILE, ROWS_PER_TILE)
    @pl.when(c == 0)
    def _():
        pltpu.sync_copy(acc_sh.at[rows], agg0_hbm.at[rows])
    @pl.when(c == 1)
    def _():
        pltpu.sync_copy(acc_sh.at[rows], agg1_hbm.at[rows])


_sc_msg = functools.partial(
    pl.kernel,
    out_type=(
        jax.ShapeDtypeStruct((NPAD, HALF), jnp.float32),
        jax.ShapeDtypeStruct((NPAD, HALF), jnp.float32),
    ),
    mesh=_MESH,
    scratch_types=[
        pltpu.VMEM((BATCHM, BLKM), jnp.int32),
        pltpu.VMEM((BATCHM, BLKM), jnp.int32),
        pltpu.VMEM((BLKM, HALF), jnp.float32),
        pltpu.VMEM((BLKM, HALF), jnp.float32),
        pltpu.VMEM_SHARED((NPAD, HALF), jnp.float32),
        pltpu.SemaphoreType.DMA,
        pltpu.SemaphoreType.DMA,
        pltpu.SemaphoreType.DMA,
        pltpu.SemaphoreType.DMA,
    ],
)(_msg_body)


_RB = 1000
_GRID = N_NODES // _RB


def _fc1_kernel(x_ref, w_ref, b_ref, d0_ref, d1_ref,
                h0_ref, g0_ref, g1_ref, dinv_ref):
    h = jnp.maximum(
        jnp.dot(x_ref[...], w_ref[...], preferred_element_type=jnp.float32)
        + b_ref[...], 0.0)
    deg = 1.0 + d0_ref[:, 0:1] + d1_ref[:, 0:1]
    dinv = lax.rsqrt(deg)
    h0_ref[...] = h
    g0_ref[...] = dinv * h[:, :HALF]
    g1_ref[...] = dinv * h[:, HALF:]
    dinv_ref[...] = jnp.broadcast_to(dinv, (_RB, HALF))


def _tc_fc1(x, fc1_w, fc1_b, deg0, deg1):
    return pl.pallas_call(
        _fc1_kernel,
        grid=(_GRID,),
        in_specs=[
            pl.BlockSpec((_RB, DIM_NODE), lambda b: (b, 0)),
            pl.BlockSpec((DIM_NODE, DIM_HIDDEN), lambda b: (0, 0)),
            pl.BlockSpec((1, DIM_HIDDEN), lambda b: (0, 0)),
            pl.BlockSpec((_RB, DEGW), lambda b: (b, 0)),
            pl.BlockSpec((_RB, DEGW), lambda b: (b, 0)),
        ],
        out_specs=[
            pl.BlockSpec((_RB, DIM_HIDDEN), lambda b: (b, 0)),
            pl.BlockSpec((_RB, HALF), lambda b: (b, 0)),
            pl.BlockSpec((_RB, HALF), lambda b: (b, 0)),
            pl.BlockSpec((_RB, HALF), lambda b: (b, 0)),
        ],
        out_shape=[
            jax.ShapeDtypeStruct((N_NODES, DIM_HIDDEN), jnp.float32),
            jax.ShapeDtypeStruct((N_NODES, HALF), jnp.float32),
            jax.ShapeDtypeStruct((N_NODES, HALF), jnp.float32),
            jax.ShapeDtypeStruct((N_NODES, HALF), jnp.float32),
        ],
    )(x, fc1_w, fc1_b, deg0, deg1)


def _layer_kernel(beta, ag0_ref, ag1_ref, g0_ref, g1_ref, h0_ref, dinv_ref,
                  w_ref, hn_ref, g0n_ref, g1n_ref):
    dinv = dinv_ref[...]
    a0 = dinv * (ag0_ref[...] + g0_ref[...])
    a1 = dinv * (ag1_ref[...] + g1_ref[...])
    z = (1.0 - ALPHA) * jnp.concatenate([a0, a1], axis=1) + ALPHA * h0_ref[...]
    out = (1.0 - beta) * z + beta * jnp.dot(
        z, w_ref[...], preferred_element_type=jnp.float32)
    h = jnp.maximum(out, 0.0)
    hn_ref[...] = h
    g0n_ref[...] = dinv * h[:, :HALF]
    g1n_ref[...] = dinv * h[:, HALF:]


def _tc_layer(beta, ag0, ag1, g0, g1, h0, dinv, w):
    return pl.pallas_call(
        functools.partial(_layer_kernel, beta),
        grid=(_GRID,),
        in_specs=[
            pl.BlockSpec((_RB, HALF), lambda b: (b, 0)),
            pl.BlockSpec((_RB, HALF), lambda b: (b, 0)),
            pl.BlockSpec((_RB, HALF), lambda b: (b, 0)),
            pl.BlockSpec((_RB, HALF), lambda b: (b, 0)),
            pl.BlockSpec((_RB, DIM_HIDDEN), lambda b: (b, 0)),
            pl.BlockSpec((_RB, HALF), lambda b: (b, 0)),
            pl.BlockSpec((DIM_HIDDEN, DIM_HIDDEN), lambda b: (0, 0)),
        ],
        out_specs=[
            pl.BlockSpec((_RB, DIM_HIDDEN), lambda b: (b, 0)),
            pl.BlockSpec((_RB, HALF), lambda b: (b, 0)),
            pl.BlockSpec((_RB, HALF), lambda b: (b, 0)),
        ],
        out_shape=[
            jax.ShapeDtypeStruct((N_NODES, DIM_HIDDEN), jnp.float32),
            jax.ShapeDtypeStruct((N_NODES, HALF), jnp.float32),
            jax.ShapeDtypeStruct((N_NODES, HALF), jnp.float32),
        ],
    )(ag0, ag1, g0, g1, h0, dinv, w)


def _fc2_kernel(h_ref, w_ref, b_ref, o_ref):
    o_ref[...] = jnp.dot(h_ref[...], w_ref[...],
                         preferred_element_type=jnp.float32) + b_ref[...]


def _tc_fc2(h, fc2_w, fc2_b):
    return pl.pallas_call(
        _fc2_kernel,
        grid=(_GRID,),
        in_specs=[
            pl.BlockSpec((_RB, DIM_HIDDEN), lambda b: (b, 0)),
            pl.BlockSpec((DIM_HIDDEN, NUM_CLASSES), lambda b: (0, 0)),
            pl.BlockSpec((1, NUM_CLASSES), lambda b: (0, 0)),
        ],
        out_specs=pl.BlockSpec((_RB, NUM_CLASSES), lambda b: (b, 0)),
        out_shape=jax.ShapeDtypeStruct((N_NODES, NUM_CLASSES), jnp.float32),
    )(h, fc2_w, fc2_b)


def kernel(x, edge_index, fc1_w, fc1_b, conv_ws, fc2_w, fc2_b):
    src = edge_index[0]
    dst = edge_index[1]
    npad_deg = EPAD - N_EDGES
    dst2d_deg = jnp.concatenate(
        [dst, N_NODES + jnp.arange(npad_deg, dtype=jnp.int32) % NTRASH]
    ).reshape(-1, BLK)
    npad_m = EPADM - N_EDGES
    src2d = jnp.concatenate(
        [src, jnp.arange(npad_m, dtype=jnp.int32) % N_NODES]).reshape(-1, BLKM)
    dst2d = jnp.concatenate(
        [dst, N_NODES + jnp.arange(npad_m, dtype=jnp.int32) % NTRASH]
    ).reshape(-1, BLKM)

    deg0, deg1 = _sc_deg(dst2d_deg)
    h0, g0, g1, dinv = _tc_fc1(x, fc1_w, fc1_b.reshape(1, -1), deg0, deg1)
    hn = h0
    for i in range(NUM_LAYERS):
        beta = log(THETA / (i + 1) + 1.0)
        ag0, ag1 = _sc_msg(g0, g1, src2d, dst2d)
        hn, g0, g1 = _tc_layer(beta, ag0, ag1, g0, g1, h0, dinv, conv_ws[i])
    return _tc_fc2(hn, fc2_w, fc2_b.reshape(1, -1))
